# Initial kernel scaffold; baseline (speedup 1.0000x reference)
import jax
import jax.numpy as jnp
from jax.experimental import pallas as pl


def kernel(sequence, pos_table):
    raise NotImplementedError("write your pallas kernel here")


# speedup vs baseline: 12.1936x; 12.1936x over previous
import jax
import jax.numpy as jnp
from jax.experimental import pallas as pl


def _broadcast_body(vec_ref, out_ref):
    out_ref[...] = jnp.broadcast_to(vec_ref[...], out_ref.shape)


def kernel(sequence, pos_table):
    batch, seq_len = sequence.shape
    hidden = pos_table.shape[1]
    flat = pos_table[:seq_len].reshape(1, seq_len * hidden)
    bb = 128
    out = pl.pallas_call(
        _broadcast_body,
        grid=(batch // bb,),
        in_specs=[pl.BlockSpec((1, seq_len * hidden), lambda i: (0, 0))],
        out_specs=pl.BlockSpec((bb, seq_len * hidden), lambda i: (i, 0)),
        out_shape=jax.ShapeDtypeStruct((batch, seq_len * hidden), jnp.float32),
    )(flat)
    return out.reshape(batch, seq_len, hidden)
